# baseline (speedup 1.0000x reference)
import functools

import jax
import jax.numpy as jnp
from jax import lax
from jax.experimental import pallas as pl
from jax.experimental.pallas import tpu as pltpu
from jax.experimental.pallas import tpu_sc as plsc

B = 16384
T = 12
E = 32
NF = 28
H = 64

NC = 2
NS = 16
NW = NC * NS
APW = B // NW

CA = 128
ROWS = CA * T
IDX_W = 128
NGATHER = ROWS // IDX_W
NCHUNK = APW // CA


V = 100000
TCH = 768
NTC = -(-V // TCH)
SLOTS = -(-NTC // NW)


def _sc_table_transpose(table_t):
    mesh = plsc.VectorSubcoreMesh(core_axis_name="c", subcore_axis_name="s")

    @functools.partial(
        pl.kernel,
        mesh=mesh,
        out_type=jax.ShapeDtypeStruct((V, E), jnp.float32),
        scratch_types=[
            pltpu.VMEM((2, E, TCH), jnp.float32),
            pltpu.VMEM((2, TCH, E), jnp.float32),
            pltpu.SemaphoreType.DMA,
        ],
        compiler_params=pltpu.CompilerParams(use_tc_tiling_on_sc=False,
                                             needs_layout_passes=False),
    )
    def body(tt_hbm, out_hbm, st_v, tr_v, osem):
        wid = lax.axis_index("s") * NC + lax.axis_index("c")
        lane16 = lax.broadcasted_iota(jnp.int32, (16,), 0)

        def tok_of(k):
            c = jnp.minimum(k * NW + wid, NTC - 1)
            return jnp.minimum(c * TCH, V - TCH)

        def stage(k, buf):
            pltpu.sync_copy(tt_hbm.at[:, pl.ds(tok_of(k), TCH)], st_v.at[buf])

        def transpose(buf):
            def group(g, inner):
                row = lane16 + g * 16
                for e in range(E):
                    plsc.store_scatter(tr_v.at[buf], [row, lane16 * 0 + e],
                                       st_v[buf, e, pl.ds(g * 16, 16)])
                return inner

            lax.fori_loop(0, TCH // 16, group, 0)

        stage(0, 0)
        handles = [None, None]
        for k in range(SLOTS):
            if k + 1 < SLOTS:
                stage(k + 1, (k + 1) % 2)
            if handles[k % 2] is not None:
                handles[k % 2].wait()
            transpose(k % 2)
            handles[k % 2] = pltpu.async_copy(
                tr_v.at[k % 2],
                out_hbm.at[pl.ds(tok_of(k), TCH)],
                osem,
            )
        for h in handles:
            if h is not None:
                h.wait()

    return body(table_t)


def _sc_embed_bag(idx_flat, table):
    mesh = plsc.VectorSubcoreMesh(core_axis_name="c", subcore_axis_name="s")

    @functools.partial(
        pl.kernel,
        mesh=mesh,
        out_type=jax.ShapeDtypeStruct((E, B), jnp.float32),
        scratch_types=[
            pltpu.VMEM((2, NGATHER, IDX_W), jnp.int32),
            pltpu.VMEM((2 * ROWS, E), jnp.float32),
            pltpu.VMEM((2, E, CA), jnp.float32),
            pltpu.SemaphoreType.DMA,
            pltpu.SemaphoreType.DMA,
        ],
        compiler_params=pltpu.CompilerParams(use_tc_tiling_on_sc=False,
                                             needs_layout_passes=False),
    )
    def body(idx_hbm, table_hbm, out_hbm, idx_v, rows_v, pooled_v, sem, osem):
        wid = lax.axis_index("s") * NC + lax.axis_index("c")
        idx_row_base = wid * (APW * T // IDX_W)
        dim_lo = lax.broadcasted_iota(jnp.int32, (16,), 0)
        dim_hi = dim_lo + 16

        def fire(c, buf):
            pltpu.sync_copy(
                idx_hbm.at[pl.ds(idx_row_base + c * NGATHER, NGATHER)],
                idx_v.at[buf])
            return [
                pltpu.async_copy(
                    table_hbm.at[idx_v.at[buf, j]],
                    rows_v.at[pl.ds(buf * ROWS + j * IDX_W, IDX_W)],
                    sem,
                )
                for j in range(NGATHER)
            ]

        def compute(buf):
            r0 = buf * ROWS

            def action_body(a, inner):
                for u in range(4):
                    bag = a * 4 + u
                    r = r0 + bag * T
                    acc0 = rows_v[r, pl.ds(0, 16)]
                    acc1 = rows_v[r, pl.ds(16, 16)]
                    for t in range(1, T):
                        acc0 = acc0 + rows_v[r + t, pl.ds(0, 16)]
                        acc1 = acc1 + rows_v[r + t, pl.ds(16, 16)]
                    col = dim_lo * 0 + bag
                    plsc.store_scatter(pooled_v.at[buf], [dim_lo, col],
                                       acc0 * (1.0 / T))
                    plsc.store_scatter(pooled_v.at[buf], [dim_hi, col],
                                       acc1 * (1.0 / T))
                return inner

            lax.fori_loop(0, CA // 4, action_body, 0)

        gath = fire(0, 0)
        out_handles = [None, None]
        for c in range(NCHUNK):
            nxt = fire(c + 1, (c + 1) % 2) if c + 1 < NCHUNK else []
            for h in gath:
                h.wait()
            if out_handles[c % 2] is not None:
                out_handles[c % 2].wait()
            compute(c % 2)
            out_handles[c % 2] = pltpu.async_copy(
                pooled_v.at[c % 2],
                out_hbm.at[:, pl.ds(wid * APW + c * CA, CA)],
                osem,
            )
            gath = nxt
        for h in out_handles:
            if h is not None:
                h.wait()

    return body(idx_flat, table)


_BM = 2048
_full = lambda shape: pl.BlockSpec(shape, lambda i: tuple(0 for _ in shape))


def _tc_numeric(numeric, W1, b1, W2, b2, Wo1b, bo1):
    def body(num_ref, W1_ref, b1_ref, W2_ref, b2_ref, Wo1b_ref, bo1_ref,
             out_ref):
        h = jnp.maximum(
            jnp.dot(num_ref[...], W1_ref[...],
                    preferred_element_type=jnp.float32) + b1_ref[...], 0.0)
        ne = jnp.dot(h, W2_ref[...],
                     preferred_element_type=jnp.float32) + b2_ref[...]
        out_ref[...] = jnp.dot(ne, Wo1b_ref[...],
                               preferred_element_type=jnp.float32) + bo1_ref[...]

    return pl.pallas_call(
        body,
        grid=(B // _BM,),
        in_specs=[
            pl.BlockSpec((_BM, NF), lambda i: (i, 0)),
            _full((NF, H)), _full((1, H)), _full((H, E)), _full((1, E)),
            _full((E, E)), _full((1, E)),
        ],
        out_specs=pl.BlockSpec((_BM, E), lambda i: (i, 0)),
        out_shape=jax.ShapeDtypeStruct((B, E), jnp.float32),
    )(numeric, W1, b1, W2, b2, Wo1b, bo1)


def _tc_head(token_embed_t, pre, Wo1a, Wo2, bo2):
    def body(te_ref, pre_ref, Wo1a_ref, Wo2_ref, bo2_ref, out_ref):
        te_blk = lax.dot_general(te_ref[...], Wo1a_ref[...],
                                 (((0,), (0,)), ((), ())),
                                 preferred_element_type=jnp.float32)
        o = jnp.maximum(te_blk + pre_ref[...], 0.0)
        out_ref[...] = lax.dot_general(Wo2_ref[...], o,
                                       (((0,), (1,)), ((), ())),
                                       preferred_element_type=jnp.float32) \
            + bo2_ref[...]

    return pl.pallas_call(
        body,
        grid=(B // _BM,),
        in_specs=[
            pl.BlockSpec((E, _BM), lambda i: (0, i)),
            pl.BlockSpec((_BM, E), lambda i: (i, 0)),
            _full((E, E)), _full((E, E)), _full((E, 1)),
        ],
        out_specs=pl.BlockSpec((E, _BM), lambda i: (0, i)),
        out_shape=jax.ShapeDtypeStruct((E, B), jnp.float32),
    )(token_embed_t, pre, Wo1a, Wo2, bo2)


def kernel(token_ids, numeric, table, W1, b1, W2, b2, Wo1, bo1, Wo2, bo2):
    idx2d = token_ids.astype(jnp.int32).reshape(B * T // IDX_W, IDX_W)
    table_rm = _sc_table_transpose(table.T)
    token_embed_t = _sc_embed_bag(idx2d, table_rm)
    pre = _tc_numeric(numeric, W1, b1.reshape(1, H), W2, b2.reshape(1, E),
                      Wo1[E:], bo1.reshape(1, E))
    out_t = _tc_head(token_embed_t, pre, Wo1[:E], Wo2, bo2.reshape(E, 1))
    return out_t.T


# speedup vs baseline: 1.2177x; 1.2177x over previous
import functools

import jax
import jax.numpy as jnp
from jax import lax
from jax.experimental import pallas as pl
from jax.experimental.pallas import tpu as pltpu
from jax.experimental.pallas import tpu_sc as plsc

B = 16384
T = 12
E = 32
NF = 28
H = 64

NC = 2
NS = 16
NW = NC * NS
APW = B // NW

CA = 128
ROWS = CA * T
IDX_W = 128
NGATHER = ROWS // IDX_W
NCHUNK = APW // CA


V = 100000
TCH = 768
NTC = -(-V // TCH)
SLOTS = -(-NTC // NW)


def _sc_table_transpose(table_t):
    mesh = plsc.VectorSubcoreMesh(core_axis_name="c", subcore_axis_name="s")

    @functools.partial(
        pl.kernel,
        mesh=mesh,
        out_type=jax.ShapeDtypeStruct((V, E), jnp.float32),
        scratch_types=[
            pltpu.VMEM((2, E, TCH), jnp.float32),
            pltpu.VMEM((2, TCH, E + 1), jnp.float32),
            pltpu.SemaphoreType.DMA,
        ],
        compiler_params=pltpu.CompilerParams(use_tc_tiling_on_sc=False,
                                             needs_layout_passes=False),
    )
    def body(tt_hbm, out_hbm, st_v, tr_v, osem):
        wid = lax.axis_index("s") * NC + lax.axis_index("c")
        lane16 = lax.broadcasted_iota(jnp.int32, (16,), 0)

        def tok_of(k):
            c = jnp.minimum(k * NW + wid, NTC - 1)
            return jnp.minimum(c * TCH, V - TCH)

        def stage(k, buf):
            pltpu.sync_copy(tt_hbm.at[:, pl.ds(tok_of(k), TCH)], st_v.at[buf])

        def transpose(buf):
            def group(g, inner):
                row = lane16 + g * 16
                for e in range(E):
                    plsc.store_scatter(tr_v.at[buf], [row, lane16 * 0 + e],
                                       st_v[buf, e, pl.ds(g * 16, 16)])
                return inner

            lax.fori_loop(0, TCH // 16, group, 0)

        stage(0, 0)
        handles = [None, None]
        for k in range(SLOTS):
            if k + 1 < SLOTS:
                stage(k + 1, (k + 1) % 2)
            if handles[k % 2] is not None:
                handles[k % 2].wait()
            transpose(k % 2)
            handles[k % 2] = pltpu.async_copy(
                tr_v.at[k % 2, :, pl.ds(0, E)],
                out_hbm.at[pl.ds(tok_of(k), TCH)],
                osem,
            )
        for h in handles:
            if h is not None:
                h.wait()

    return body(table_t)


def _sc_embed_bag(idx_flat, table):
    mesh = plsc.VectorSubcoreMesh(core_axis_name="c", subcore_axis_name="s")

    @functools.partial(
        pl.kernel,
        mesh=mesh,
        out_type=jax.ShapeDtypeStruct((E, B), jnp.float32),
        scratch_types=[
            pltpu.VMEM((2, NGATHER, IDX_W), jnp.int32),
            pltpu.VMEM((2 * ROWS, E), jnp.float32),
            pltpu.VMEM((2, E, CA), jnp.float32),
            pltpu.SemaphoreType.DMA,
            pltpu.SemaphoreType.DMA,
        ],
        compiler_params=pltpu.CompilerParams(use_tc_tiling_on_sc=False,
                                             needs_layout_passes=False),
    )
    def body(idx_hbm, table_hbm, out_hbm, idx_v, rows_v, pooled_v, sem, osem):
        wid = lax.axis_index("s") * NC + lax.axis_index("c")
        idx_row_base = wid * (APW * T // IDX_W)
        dim_lo = lax.broadcasted_iota(jnp.int32, (16,), 0)
        dim_hi = dim_lo + 16

        def fire(c, buf):
            pltpu.sync_copy(
                idx_hbm.at[pl.ds(idx_row_base + c * NGATHER, NGATHER)],
                idx_v.at[buf])
            return [
                pltpu.async_copy(
                    table_hbm.at[idx_v.at[buf, j]],
                    rows_v.at[pl.ds(buf * ROWS + j * IDX_W, IDX_W)],
                    sem,
                )
                for j in range(NGATHER)
            ]

        def compute(buf):
            r0 = buf * ROWS

            def action_body(a, inner):
                for u in range(4):
                    bag = a * 4 + u
                    r = r0 + bag * T
                    acc0 = rows_v[r, pl.ds(0, 16)]
                    acc1 = rows_v[r, pl.ds(16, 16)]
                    for t in range(1, T):
                        acc0 = acc0 + rows_v[r + t, pl.ds(0, 16)]
                        acc1 = acc1 + rows_v[r + t, pl.ds(16, 16)]
                    col = dim_lo * 0 + bag
                    plsc.store_scatter(pooled_v.at[buf], [dim_lo, col],
                                       acc0 * (1.0 / T))
                    plsc.store_scatter(pooled_v.at[buf], [dim_hi, col],
                                       acc1 * (1.0 / T))
                return inner

            lax.fori_loop(0, CA // 4, action_body, 0)

        gath = fire(0, 0)
        out_handles = [None, None]
        for c in range(NCHUNK):
            nxt = fire(c + 1, (c + 1) % 2) if c + 1 < NCHUNK else []
            for h in gath:
                h.wait()
            if out_handles[c % 2] is not None:
                out_handles[c % 2].wait()
            compute(c % 2)
            out_handles[c % 2] = pltpu.async_copy(
                pooled_v.at[c % 2],
                out_hbm.at[:, pl.ds(wid * APW + c * CA, CA)],
                osem,
            )
            gath = nxt
        for h in out_handles:
            if h is not None:
                h.wait()

    return body(idx_flat, table)


_BM = 2048
_full = lambda shape: pl.BlockSpec(shape, lambda i: tuple(0 for _ in shape))


def _tc_numeric(numeric, W1, b1, W2, b2, Wo1b, bo1):
    def body(num_ref, W1_ref, b1_ref, W2_ref, b2_ref, Wo1b_ref, bo1_ref,
             out_ref):
        h = jnp.maximum(
            jnp.dot(num_ref[...], W1_ref[...],
                    preferred_element_type=jnp.float32) + b1_ref[...], 0.0)
        ne = jnp.dot(h, W2_ref[...],
                     preferred_element_type=jnp.float32) + b2_ref[...]
        out_ref[...] = jnp.dot(ne, Wo1b_ref[...],
                               preferred_element_type=jnp.float32) + bo1_ref[...]

    return pl.pallas_call(
        body,
        grid=(B // _BM,),
        in_specs=[
            pl.BlockSpec((_BM, NF), lambda i: (i, 0)),
            _full((NF, H)), _full((1, H)), _full((H, E)), _full((1, E)),
            _full((E, E)), _full((1, E)),
        ],
        out_specs=pl.BlockSpec((_BM, E), lambda i: (i, 0)),
        out_shape=jax.ShapeDtypeStruct((B, E), jnp.float32),
    )(numeric, W1, b1, W2, b2, Wo1b, bo1)


def _tc_head(token_embed_t, pre, Wo1a, Wo2, bo2):
    def body(te_ref, pre_ref, Wo1a_ref, Wo2_ref, bo2_ref, out_ref):
        te_blk = lax.dot_general(te_ref[...], Wo1a_ref[...],
                                 (((0,), (0,)), ((), ())),
                                 preferred_element_type=jnp.float32)
        o = jnp.maximum(te_blk + pre_ref[...], 0.0)
        out_ref[...] = lax.dot_general(Wo2_ref[...], o,
                                       (((0,), (1,)), ((), ())),
                                       preferred_element_type=jnp.float32) \
            + bo2_ref[...]

    return pl.pallas_call(
        body,
        grid=(B // _BM,),
        in_specs=[
            pl.BlockSpec((E, _BM), lambda i: (0, i)),
            pl.BlockSpec((_BM, E), lambda i: (i, 0)),
            _full((E, E)), _full((E, E)), _full((E, 1)),
        ],
        out_specs=pl.BlockSpec((E, _BM), lambda i: (0, i)),
        out_shape=jax.ShapeDtypeStruct((E, B), jnp.float32),
    )(token_embed_t, pre, Wo1a, Wo2, bo2)


def kernel(token_ids, numeric, table, W1, b1, W2, b2, Wo1, bo1, Wo2, bo2):
    idx2d = token_ids.astype(jnp.int32).reshape(B * T // IDX_W, IDX_W)
    table_rm = _sc_table_transpose(table.T)
    token_embed_t = _sc_embed_bag(idx2d, table_rm)
    pre = _tc_numeric(numeric, W1, b1.reshape(1, H), W2, b2.reshape(1, E),
                      Wo1[E:], bo1.reshape(1, E))
    out_t = _tc_head(token_embed_t, pre, Wo1[:E], Wo2, bo2.reshape(E, 1))
    return out_t.T


# speedup vs baseline: 1.5787x; 1.2965x over previous
import functools

import jax
import jax.numpy as jnp
from jax import lax
from jax.experimental import pallas as pl
from jax.experimental.pallas import tpu as pltpu
from jax.experimental.pallas import tpu_sc as plsc

B = 16384
T = 12
E = 32
NF = 28
H = 64

NC = 2
NS = 16
NW = NC * NS
APW = B // NW

CA = 128
ROWS = CA * T
IDX_W = 128
NGATHER = ROWS // IDX_W
NCHUNK = APW // CA


def _sc_embed_bag(idx_t, table):
    mesh = plsc.VectorSubcoreMesh(core_axis_name="c", subcore_axis_name="s")

    @functools.partial(
        pl.kernel,
        mesh=mesh,
        out_type=jax.ShapeDtypeStruct((E, B), jnp.float32),
        scratch_types=[
            pltpu.VMEM((2, T, CA), jnp.int32),
            pltpu.VMEM((2 * ROWS, E), jnp.float32),
            pltpu.VMEM((2, E, CA), jnp.float32),
            pltpu.SemaphoreType.DMA,
            pltpu.SemaphoreType.DMA,
        ],
        compiler_params=pltpu.CompilerParams(use_tc_tiling_on_sc=False,
                                             needs_layout_passes=False),
    )
    def body(idx_hbm, table_hbm, out_hbm, idx_v, rows_v, pooled_v, sem, osem):
        wid = lax.axis_index("s") * NC + lax.axis_index("c")
        dim_lo = lax.broadcasted_iota(jnp.int32, (16,), 0)
        dim_hi = dim_lo + 16

        def fire(c, buf):
            bag0 = wid * APW + c * CA
            pltpu.sync_copy(idx_hbm.at[:, pl.ds(bag0, CA)], idx_v.at[buf])
            return [
                pltpu.async_copy(
                    table_hbm.at[idx_v.at[buf, t]],
                    rows_v.at[pl.ds(buf * ROWS + t * CA, CA)],
                    sem,
                )
                for t in range(T)
            ]

        def compute(buf):
            r0 = buf * ROWS

            def action_body(a, inner):
                for u in range(4):
                    bag = a * 4 + u
                    r = r0 + bag
                    acc0 = rows_v[r, pl.ds(0, 16)]
                    acc1 = rows_v[r, pl.ds(16, 16)]
                    for t in range(1, T):
                        acc0 = acc0 + rows_v[r + t * CA, pl.ds(0, 16)]
                        acc1 = acc1 + rows_v[r + t * CA, pl.ds(16, 16)]
                    col = dim_lo * 0 + bag
                    plsc.store_scatter(pooled_v.at[buf], [dim_lo, col],
                                       acc0 * (1.0 / T))
                    plsc.store_scatter(pooled_v.at[buf], [dim_hi, col],
                                       acc1 * (1.0 / T))
                return inner

            lax.fori_loop(0, CA // 4, action_body, 0)

        gath = fire(0, 0)
        out_handles = [None, None]
        for c in range(NCHUNK):
            nxt = fire(c + 1, (c + 1) % 2) if c + 1 < NCHUNK else []
            for h in gath:
                h.wait()
            if out_handles[c % 2] is not None:
                out_handles[c % 2].wait()
            compute(c % 2)
            out_handles[c % 2] = pltpu.async_copy(
                pooled_v.at[c % 2],
                out_hbm.at[:, pl.ds(wid * APW + c * CA, CA)],
                osem,
            )
            gath = nxt
        for h in out_handles:
            if h is not None:
                h.wait()

    return body(idx_t, table)


_BM = 2048
_full = lambda shape: pl.BlockSpec(shape, lambda i: tuple(0 for _ in shape))


def _tc_numeric(numeric, W1, b1, W2, b2, Wo1b, bo1):
    def body(num_ref, W1_ref, b1_ref, W2_ref, b2_ref, Wo1b_ref, bo1_ref,
             out_ref):
        h = jnp.maximum(
            jnp.dot(num_ref[...], W1_ref[...],
                    preferred_element_type=jnp.float32) + b1_ref[...], 0.0)
        ne = jnp.dot(h, W2_ref[...],
                     preferred_element_type=jnp.float32) + b2_ref[...]
        out_ref[...] = jnp.dot(ne, Wo1b_ref[...],
                               preferred_element_type=jnp.float32) + bo1_ref[...]

    return pl.pallas_call(
        body,
        grid=(B // _BM,),
        in_specs=[
            pl.BlockSpec((_BM, NF), lambda i: (i, 0)),
            _full((NF, H)), _full((1, H)), _full((H, E)), _full((1, E)),
            _full((E, E)), _full((1, E)),
        ],
        out_specs=pl.BlockSpec((_BM, E), lambda i: (i, 0)),
        out_shape=jax.ShapeDtypeStruct((B, E), jnp.float32),
    )(numeric, W1, b1, W2, b2, Wo1b, bo1)


def _tc_head(token_embed_t, pre, Wo1a, Wo2, bo2):
    def body(te_ref, pre_ref, Wo1a_ref, Wo2_ref, bo2_ref, out_ref):
        te_blk = lax.dot_general(te_ref[...], Wo1a_ref[...],
                                 (((0,), (0,)), ((), ())),
                                 preferred_element_type=jnp.float32)
        o = jnp.maximum(te_blk + pre_ref[...], 0.0)
        out_ref[...] = lax.dot_general(Wo2_ref[...], o,
                                       (((0,), (1,)), ((), ())),
                                       preferred_element_type=jnp.float32) \
            + bo2_ref[...]

    return pl.pallas_call(
        body,
        grid=(B // _BM,),
        in_specs=[
            pl.BlockSpec((E, _BM), lambda i: (0, i)),
            pl.BlockSpec((_BM, E), lambda i: (i, 0)),
            _full((E, E)), _full((E, E)), _full((E, 1)),
        ],
        out_specs=pl.BlockSpec((E, _BM), lambda i: (0, i)),
        out_shape=jax.ShapeDtypeStruct((E, B), jnp.float32),
    )(token_embed_t, pre, Wo1a, Wo2, bo2)


def kernel(token_ids, numeric, table, W1, b1, W2, b2, Wo1, bo1, Wo2, bo2):
    token_embed_t = _sc_embed_bag(token_ids.astype(jnp.int32).T, table)
    pre = _tc_numeric(numeric, W1, b1.reshape(1, H), W2, b2.reshape(1, E),
                      Wo1[E:], bo1.reshape(1, E))
    out_t = _tc_head(token_embed_t, pre, Wo1[:E], Wo2, bo2.reshape(E, 1))
    return out_t.T
